# baseline (speedup 1.0000x reference)
import numpy as np
import jax
import jax.numpy as jnp
from jax import lax
from jax.experimental import pallas as pl
from jax.experimental.pallas import tpu as pltpu
from jax.experimental.pallas import tpu_sc as plsc

B, H, W, NCLS = 2, 480, 640, 22
HW = H * W
NPIX = B * HW
RH = B * H
NC, NS = 2, 16
NW = NC * NS
NUNIT = RH // 8
JROW = W // 16
NBIN = B * (NCLS - 1)
ACCW = NBIN * 8
ACCL = ACCW + 1
VOTESTRIDE = 48
VOTEL = VOTESTRIDE + 1
BH = 16
EPS = 1e-6
F32 = jnp.float32


def _rsqrt(n2):
    i = lax.bitcast_convert_type(n2, jnp.int32)
    i = 0x5F3759DF - (i >> 1)
    r = lax.bitcast_convert_type(i, F32)
    for _ in range(3):
        r = r * (1.5 - 0.5 * n2 * r * r)
    return r


def _sel_body(lab_ref, msk_ref, vp_ref, labe_ref, dxm_ref, dym_ref, vzm_ref):
    lab = lab_ref[...]
    msk = msk_ref[...]
    le = jnp.where((msk > 0) & (lab > 0), lab, 0)
    vx = vp_ref[0, 0]
    vy = vp_ref[0, 1]
    vz = vp_ref[0, 2]
    for c in range(1, NCLS):
        mc = le == c
        vx = jnp.where(mc, vp_ref[0, 3 * c], vx)
        vy = jnp.where(mc, vp_ref[0, 3 * c + 1], vy)
        vz = jnp.where(mc, vp_ref[0, 3 * c + 2], vz)
    valid = le > 0
    n2 = jnp.maximum(vx * vx + vy * vy, 1e-12)
    r = lax.rsqrt(n2)
    s = r / (1.0 + EPS * r)
    zero = jnp.zeros_like(vx)
    labe_ref[...] = le
    dxm_ref[...] = jnp.where(valid, vx * s, zero)
    dym_ref[...] = jnp.where(valid, vy * s, zero)
    vzm_ref[...] = jnp.where(valid, vz, zero)


_plane_spec = pl.BlockSpec((BH, W), lambda g: (g, 0))

_select = pl.pallas_call(
    _sel_body,
    grid=(RH // BH,),
    in_specs=[
        _plane_spec,
        _plane_spec,
        pl.BlockSpec((1, 3 * NCLS, BH, W),
                     lambda g: (g // (H // BH), 0, g % (H // BH), 0)),
    ],
    out_specs=[_plane_spec, _plane_spec, _plane_spec, _plane_spec],
    out_shape=[
        jax.ShapeDtypeStruct((RH, W), jnp.int32),
        jax.ShapeDtypeStruct((RH, W), F32),
        jax.ShapeDtypeStruct((RH, W), F32),
        jax.ShapeDtypeStruct((RH, W), F32),
    ],
)


def _mom_body(labe_hbm, dxm_hbm, dym_hbm, vzm_hbm, part_hbm,
              le_v, dx_v, dy_v, vz_v, acc_v):
    cid = lax.axis_index("c")
    sid = lax.axis_index("s")
    wid = cid * NS + sid
    b = wid // NS
    u0 = (wid * NUNIT) // NW
    u1 = ((wid + 1) * NUNIT) // NW
    lanes = lax.iota(jnp.int32, 16)

    def _zero(i, _):
        acc_v[pl.ds(i * 16, 16)] = jnp.zeros((16,), F32)
        return 0
    lax.fori_loop(0, (16 * ACCL + 15) // 16, _zero, 0)

    def _unit(u, _):
        r0 = u * 8
        pltpu.sync_copy(labe_hbm.at[pl.ds(r0, 8)], le_v)
        pltpu.sync_copy(dxm_hbm.at[pl.ds(r0, 8)], dx_v)
        pltpu.sync_copy(dym_hbm.at[pl.ds(r0, 8)], dy_v)
        pltpu.sync_copy(vzm_hbm.at[pl.ds(r0, 8)], vz_v)

        def _row(rr, carry):
            yf = (r0 + rr - b * H).astype(F32)

            def _grp(j4, cc):
                for k in range(4):
                    cs = j4 * 64 + k * 16
                    le = le_v[rr, pl.ds(cs, 16)]
                    dx = dx_v[rr, pl.ds(cs, 16)]
                    dy = dy_v[rr, pl.ds(cs, 16)]
                    vz = vz_v[rr, pl.ds(cs, 16)]
                    valid = le > 0
                    xf = (cs + lanes).astype(F32)
                    t1 = 1.0 - dx * dx
                    t2 = -dx * dy
                    t3 = 1.0 - dy * dy
                    s1 = t1 * xf + t2 * yf
                    s2 = t2 * xf + t3 * yf
                    binv = b * (NCLS - 1) + jnp.maximum(le - 1, 0)
                    ab = lanes * ACCL + binv * 8
                    one = jnp.ones((16,), F32)
                    plsc.addupdate_scatter(acc_v, [ab], one, mask=valid)
                    plsc.addupdate_scatter(acc_v, [ab + 1], t1, mask=valid)
                    plsc.addupdate_scatter(acc_v, [ab + 2], t2, mask=valid)
                    plsc.addupdate_scatter(acc_v, [ab + 3], t3, mask=valid)
                    plsc.addupdate_scatter(acc_v, [ab + 4], s1, mask=valid)
                    plsc.addupdate_scatter(acc_v, [ab + 5], s2, mask=valid)
                    plsc.addupdate_scatter(acc_v, [ab + 6], vz, mask=valid)
                return cc
            lax.fori_loop(0, JROW // 4, _grp, 0)
            return carry
        lax.fori_loop(0, 8, _row, 0)
        return 0
    lax.fori_loop(u0, u1, _unit, 0)

    def _fold(t, _):
        v = acc_v[pl.ds(t * 16, 16)]
        for l in range(1, 16):
            v = v + acc_v[pl.ds(l * ACCL + t * 16, 16)]
        acc_v[pl.ds(t * 16, 16)] = v
        return 0
    lax.fori_loop(0, ACCW // 16, _fold, 0)
    pltpu.sync_copy(acc_v.at[pl.ds(0, ACCW)],
                    part_hbm.at[pl.ds(wid * ACCW, ACCW)])


def _vote_body(labe_hbm, dxm_hbm, dym_hbm, part_hbm,
               votes_hbm, sums_hbm,
               le_v, dx_v, dy_v, part_v, cx_v, cy_v, vacc_v):
    cid = lax.axis_index("c")
    sid = lax.axis_index("s")
    wid = cid * NS + sid
    b = wid // NS
    u0 = (wid * NUNIT) // NW
    u1 = ((wid + 1) * NUNIT) // NW
    lanes = lax.iota(jnp.int32, 16)

    pltpu.sync_copy(part_hbm, part_v)

    def _red(t, _):
        v = part_v[pl.ds(t * 16, 16)]
        for wk in range(1, NW):
            v = v + part_v[pl.ds(wk * ACCW + t * 16, 16)]
        part_v[pl.ds(t * 16, 16)] = v
        return 0
    lax.fori_loop(0, ACCW // 16, _red, 0)

    @pl.when(wid == 0)
    def _():
        pltpu.sync_copy(part_v.at[pl.ds(0, ACCW)], sums_hbm)

    def _slv(t, _):
        binv = t * 16 + lanes
        bi8 = binv * 8
        a11 = plsc.load_gather(part_v, [bi8 + 1])
        a12 = plsc.load_gather(part_v, [bi8 + 2])
        a22 = plsc.load_gather(part_v, [bi8 + 3])
        b1 = plsc.load_gather(part_v, [bi8 + 4])
        b2 = plsc.load_gather(part_v, [bi8 + 5])
        det = a11 * a22 - a12 * a12
        det = jnp.where(jnp.abs(det) < EPS, jnp.full((16,), EPS, F32), det)
        cx = (a22 * b1 - a12 * b2) / det
        cy = (a11 * b2 - a12 * b1) / det
        for l in range(16):
            cx_v[pl.ds(l * VOTEL + t * 16, 16)] = cx
            cy_v[pl.ds(l * VOTEL + t * 16, 16)] = cy
        return 0
    lax.fori_loop(0, 3, _slv, 0)

    def _zero(i, _):
        vacc_v[pl.ds(i * 16, 16)] = jnp.zeros((16,), F32)
        return 0
    lax.fori_loop(0, (16 * VOTEL + 15) // 16, _zero, 0)

    def _unit(u, _):
        r0 = u * 8
        pltpu.sync_copy(labe_hbm.at[pl.ds(r0, 8)], le_v)
        pltpu.sync_copy(dxm_hbm.at[pl.ds(r0, 8)], dx_v)
        pltpu.sync_copy(dym_hbm.at[pl.ds(r0, 8)], dy_v)

        def _row(rr, carry):
            yf = (r0 + rr - b * H).astype(F32)

            def _grp(j4, cc):
                for k in range(4):
                    cs = j4 * 64 + k * 16
                    le = le_v[rr, pl.ds(cs, 16)]
                    dxm = dx_v[rr, pl.ds(cs, 16)]
                    dym = dy_v[rr, pl.ds(cs, 16)]
                    binv = b * (NCLS - 1) + jnp.maximum(le - 1, 0)
                    ab = lanes * VOTEL + binv
                    cx = plsc.load_gather(cx_v, [ab])
                    cy = plsc.load_gather(cy_v, [ab])
                    xf = (cs + lanes).astype(F32)
                    ux = cx - xf
                    uy = cy - yf
                    u2 = jnp.maximum(ux * ux + uy * uy, 1e-12)
                    r2 = _rsqrt(u2)
                    num = ux * dxm + uy * dym
                    vm = num > 0.9 * (u2 * r2 + EPS)
                    one = jnp.ones((16,), F32)
                    plsc.addupdate_scatter(vacc_v, [ab], one, mask=vm)
                return cc
            lax.fori_loop(0, JROW // 4, _grp, 0)
            return carry
        lax.fori_loop(0, 8, _row, 0)
        return 0
    lax.fori_loop(u0, u1, _unit, 0)

    def _foldv(t, _):
        v = vacc_v[pl.ds(t * 16, 16)]
        for l in range(1, 16):
            v = v + vacc_v[pl.ds(l * VOTEL + t * 16, 16)]
        vacc_v[pl.ds(t * 16, 16)] = v
        return 0
    lax.fori_loop(0, VOTESTRIDE // 16, _foldv, 0)
    pltpu.sync_copy(vacc_v.at[pl.ds(0, VOTESTRIDE)],
                    votes_hbm.at[pl.ds(wid * VOTESTRIDE, VOTESTRIDE)])


def _fin_body(sums_ref, votes_ref, ext_ref, poses_ref, meta_ref,
              box_ref, pose_ref):
    sums = sums_ref[...]
    votes = jnp.sum(votes_ref[...], axis=0)[:NBIN]
    cnt = sums[:, 0]
    a11 = sums[:, 1]
    a12 = sums[:, 2]
    a22 = sums[:, 3]
    b1 = sums[:, 4]
    b2 = sums[:, 5]
    sz = sums[:, 6]
    det = a11 * a22 - a12 * a12
    det = jnp.where(jnp.abs(det) < EPS, EPS, det)
    cx = (a22 * b1 - a12 * b2) / det
    cy = (a11 * b2 - a12 * b1) / det

    bsel = lax.broadcasted_iota(jnp.int32, (NBIN,), 0) >= (NCLS - 1)
    fx = jnp.where(bsel, meta_ref[1, 0], meta_ref[0, 0]) + EPS
    fy = jnp.where(bsel, meta_ref[1, 4], meta_ref[0, 4]) + EPS
    px0 = jnp.where(bsel, meta_ref[1, 2], meta_ref[0, 2])
    py0 = jnp.where(bsel, meta_ref[1, 5], meta_ref[0, 5])

    frac = votes / jnp.maximum(cnt, 1.0)
    depth = jnp.exp(sz / jnp.maximum(cnt, 1.0))
    tx = depth * (cx - px0) / fx
    ty = depth * (cy - py0) / fy

    e = ext_ref[...]
    ext2 = jnp.sum(e * e, axis=1)
    e21 = ext2[1:NCLS]
    ext2sel = jnp.concatenate([e21, e21])
    extv = jnp.sqrt(ext2sel + EPS)
    bw = 0.5 * fx * extv / (depth + EPS)
    bh = 0.5 * fy * extv / (depth + EPS)
    validv = ((cnt > 500.0) & (votes >= 100.0)).astype(F32)
    score = frac * validv

    rowi = lax.broadcasted_iota(jnp.int32, (NBIN,), 0)
    clsv = (rowi + 1 - (NCLS - 1) * bsel.astype(jnp.int32)).astype(F32)
    box_ref[0, :] = bsel.astype(F32)
    box_ref[1, :] = clsv
    box_ref[2, :] = cx - bw
    box_ref[3, :] = cy - bh
    box_ref[4, :] = cx + bw
    box_ref[5, :] = cy + bh
    box_ref[6, :] = score

    for k in range(4):
        pose_ref[k, :] = jnp.where(bsel, poses_ref[1, 6 + k],
                                   poses_ref[0, 6 + k])
    pose_ref[4, :] = tx
    pose_ref[5, :] = ty
    pose_ref[6, :] = depth


_mesh = plsc.VectorSubcoreMesh(core_axis_name="c", subcore_axis_name="s")

_moments = pl.kernel(
    _mom_body,
    out_type=[jax.ShapeDtypeStruct((NW * ACCW,), F32)],
    mesh=_mesh,
    compiler_params=pltpu.CompilerParams(needs_layout_passes=False),
    scratch_types=[
        pltpu.VMEM((8, W), jnp.int32),
        pltpu.VMEM((8, W), F32),
        pltpu.VMEM((8, W), F32),
        pltpu.VMEM((8, W), F32),
        pltpu.VMEM((16 * ACCL + 16,), F32),
    ],
)

_votes = pl.kernel(
    _vote_body,
    out_type=[
        jax.ShapeDtypeStruct((NW * VOTESTRIDE,), F32),
        jax.ShapeDtypeStruct((ACCW,), F32),
    ],
    mesh=_mesh,
    compiler_params=pltpu.CompilerParams(needs_layout_passes=False),
    scratch_types=[
        pltpu.VMEM((8, W), jnp.int32),
        pltpu.VMEM((8, W), F32),
        pltpu.VMEM((8, W), F32),
        pltpu.VMEM((NW * ACCW,), F32),
        pltpu.VMEM((16 * VOTEL + 16,), F32),
        pltpu.VMEM((16 * VOTEL + 16,), F32),
        pltpu.VMEM((16 * VOTEL + 16,), F32),
    ],
)

_final = pl.pallas_call(
    _fin_body,
    out_shape=[
        jax.ShapeDtypeStruct((7, NBIN), F32),
        jax.ShapeDtypeStruct((7, NBIN), F32),
    ],
)


def kernel(labels, masks, vertex_pred, extents, poses, meta_data):
    lab2 = labels.reshape(RH, W)
    msk2 = masks.reshape(RH, W)
    labe, dxm, dym, vzm = _select(lab2, msk2, vertex_pred)
    part, = _moments(labe, dxm, dym, vzm)
    votes, sums = _votes(labe, dxm, dym, part)
    boxt, poset = _final(sums.reshape(NBIN, 8), votes.reshape(NW, VOTESTRIDE),
                         extents, poses, meta_data)
    top_box = boxt.T
    top_pose = poset.T
    top_target = jnp.zeros((NBIN, 4 * NCLS), F32)
    top_weight = jnp.zeros((NBIN, 4 * NCLS), F32)
    top_domain = jnp.repeat(jnp.arange(B, dtype=F32), NCLS - 1)
    return top_box, top_pose, top_target, top_weight, top_domain


# speedup vs baseline: 1.0219x; 1.0219x over previous
import numpy as np
import jax
import jax.numpy as jnp
from jax import lax
from jax.experimental import pallas as pl
from jax.experimental.pallas import tpu as pltpu
from jax.experimental.pallas import tpu_sc as plsc

B, H, W, NCLS = 2, 480, 640, 22
HW = H * W
NPIX = B * HW
RH = B * H
NC, NS = 2, 16
NW = NC * NS
NUNIT = RH // 8
JROW = W // 16
NBIN = B * (NCLS - 1)
ACCW = NBIN * 8
ACCL = ACCW + 1
VOTESTRIDE = 48
VOTEL = VOTESTRIDE + 1
BH = 16
EPS = 1e-6
F32 = jnp.float32


def _rsqrt(n2):
    i = lax.bitcast_convert_type(n2, jnp.int32)
    i = 0x5F3759DF - (i >> 1)
    r = lax.bitcast_convert_type(i, F32)
    for _ in range(3):
        r = r * (1.5 - 0.5 * n2 * r * r)
    return r


def _sel_body(lab_ref, msk_ref, vp_ref, labe_ref, dxm_ref, dym_ref, vzm_ref):
    lab = lab_ref[...]
    msk = msk_ref[...]
    le = jnp.where((msk > 0) & (lab > 0), lab, 0)
    vx = vp_ref[0, 0]
    vy = vp_ref[0, 1]
    vz = vp_ref[0, 2]
    for c in range(1, NCLS):
        mc = le == c
        vx = jnp.where(mc, vp_ref[0, 3 * c], vx)
        vy = jnp.where(mc, vp_ref[0, 3 * c + 1], vy)
        vz = jnp.where(mc, vp_ref[0, 3 * c + 2], vz)
    valid = le > 0
    n2 = jnp.maximum(vx * vx + vy * vy, 1e-12)
    r = lax.rsqrt(n2)
    s = r / (1.0 + EPS * r)
    zero = jnp.zeros_like(vx)
    labe_ref[...] = le
    dxm_ref[...] = jnp.where(valid, vx * s, zero)
    dym_ref[...] = jnp.where(valid, vy * s, zero)
    vzm_ref[...] = jnp.where(valid, vz, zero)


_plane_spec = pl.BlockSpec((BH, W), lambda g: (g, 0))

_select = pl.pallas_call(
    _sel_body,
    grid=(RH // BH,),
    in_specs=[
        _plane_spec,
        _plane_spec,
        pl.BlockSpec((1, 3 * NCLS, BH, W),
                     lambda g: (g // (H // BH), 0, g % (H // BH), 0)),
    ],
    out_specs=[_plane_spec, _plane_spec, _plane_spec, _plane_spec],
    out_shape=[
        jax.ShapeDtypeStruct((RH, W), jnp.int32),
        jax.ShapeDtypeStruct((RH, W), F32),
        jax.ShapeDtypeStruct((RH, W), F32),
        jax.ShapeDtypeStruct((RH, W), F32),
    ],
)


def _mom_body(labe_hbm, dxm_hbm, dym_hbm, vzm_hbm, part_hbm,
              le_v, dx_v, dy_v, vz_v, acc_v):
    cid = lax.axis_index("c")
    sid = lax.axis_index("s")
    wid = cid * NS + sid
    b = wid // NS
    u0 = (wid * NUNIT) // NW
    u1 = ((wid + 1) * NUNIT) // NW
    lanes = lax.iota(jnp.int32, 16)

    def _zero(i, _):
        acc_v[pl.ds(i * 16, 16)] = jnp.zeros((16,), F32)
        return 0
    lax.fori_loop(0, (16 * ACCL + 15) // 16, _zero, 0)

    def _unit(u, _):
        r0 = u * 8
        pltpu.sync_copy(labe_hbm.at[pl.ds(r0, 8)], le_v)
        pltpu.sync_copy(dxm_hbm.at[pl.ds(r0, 8)], dx_v)
        pltpu.sync_copy(dym_hbm.at[pl.ds(r0, 8)], dy_v)
        pltpu.sync_copy(vzm_hbm.at[pl.ds(r0, 8)], vz_v)

        def _row(rr, carry):
            yf = (r0 + rr - b * H).astype(F32)

            def _grp(j4, cc):
                for k in range(4):
                    cs = j4 * 64 + k * 16
                    le = le_v[rr, pl.ds(cs, 16)]
                    dx = dx_v[rr, pl.ds(cs, 16)]
                    dy = dy_v[rr, pl.ds(cs, 16)]
                    vz = vz_v[rr, pl.ds(cs, 16)]
                    valid = le > 0
                    xf = (cs + lanes).astype(F32)
                    t1 = 1.0 - dx * dx
                    t2 = -dx * dy
                    t3 = 1.0 - dy * dy
                    s1 = t1 * xf + t2 * yf
                    s2 = t2 * xf + t3 * yf
                    binv = b * (NCLS - 1) + jnp.maximum(le - 1, 0)
                    ab = lanes * ACCL + binv * 8
                    one = jnp.ones((16,), F32)
                    plsc.addupdate_scatter(acc_v, [ab], one, mask=valid)
                    plsc.addupdate_scatter(acc_v, [ab + 1], t1, mask=valid)
                    plsc.addupdate_scatter(acc_v, [ab + 2], t2, mask=valid)
                    plsc.addupdate_scatter(acc_v, [ab + 3], t3, mask=valid)
                    plsc.addupdate_scatter(acc_v, [ab + 4], s1, mask=valid)
                    plsc.addupdate_scatter(acc_v, [ab + 5], s2, mask=valid)
                    plsc.addupdate_scatter(acc_v, [ab + 6], vz, mask=valid)
                return cc
            lax.fori_loop(0, JROW // 4, _grp, 0)
            return carry
        lax.fori_loop(0, 8, _row, 0)
        return 0
    lax.fori_loop(u0, u1, _unit, 0)

    def _fold(t, _):
        v = acc_v[pl.ds(t * 16, 16)]
        for l in range(1, 16):
            v = v + acc_v[pl.ds(l * ACCL + t * 16, 16)]
        acc_v[pl.ds(t * 16, 16)] = v
        return 0
    lax.fori_loop(0, ACCW // 16, _fold, 0)
    pltpu.sync_copy(acc_v.at[pl.ds(0, ACCW)],
                    part_hbm.at[pl.ds(wid * ACCW, ACCW)])


def _vote_body(labe_hbm, dxm_hbm, dym_hbm, part_hbm,
               votes_hbm, sums_hbm,
               le_v, dx_v, dy_v, part_v, cx_v, cy_v, vacc_v):
    cid = lax.axis_index("c")
    sid = lax.axis_index("s")
    wid = cid * NS + sid
    b = wid // NS
    u0 = (wid * NUNIT) // NW
    u1 = ((wid + 1) * NUNIT) // NW
    lanes = lax.iota(jnp.int32, 16)

    pltpu.sync_copy(part_hbm, part_v)

    def _red(t, _):
        v = part_v[pl.ds(t * 16, 16)]
        for wk in range(1, NW):
            v = v + part_v[pl.ds(wk * ACCW + t * 16, 16)]
        part_v[pl.ds(t * 16, 16)] = v
        return 0
    lax.fori_loop(0, ACCW // 16, _red, 0)

    @pl.when(wid == 0)
    def _():
        pltpu.sync_copy(part_v.at[pl.ds(0, ACCW)], sums_hbm)

    def _slv(t, _):
        binv = t * 16 + lanes
        bi8 = binv * 8
        a11 = plsc.load_gather(part_v, [bi8 + 1])
        a12 = plsc.load_gather(part_v, [bi8 + 2])
        a22 = plsc.load_gather(part_v, [bi8 + 3])
        b1 = plsc.load_gather(part_v, [bi8 + 4])
        b2 = plsc.load_gather(part_v, [bi8 + 5])
        det = a11 * a22 - a12 * a12
        det = jnp.where(jnp.abs(det) < EPS, jnp.full((16,), EPS, F32), det)
        cx_v[pl.ds(t * 16, 16)] = (a22 * b1 - a12 * b2) / det
        cy_v[pl.ds(t * 16, 16)] = (a11 * b2 - a12 * b1) / det
        return 0
    lax.fori_loop(0, 3, _slv, 0)

    def _zero(i, _):
        vacc_v[pl.ds(i * 16, 16)] = jnp.zeros((16,), F32)
        return 0
    lax.fori_loop(0, (16 * VOTEL + 15) // 16, _zero, 0)

    def _unit(u, _):
        r0 = u * 8
        pltpu.sync_copy(labe_hbm.at[pl.ds(r0, 8)], le_v)
        pltpu.sync_copy(dxm_hbm.at[pl.ds(r0, 8)], dx_v)
        pltpu.sync_copy(dym_hbm.at[pl.ds(r0, 8)], dy_v)

        def _row(rr, carry):
            yf = (r0 + rr - b * H).astype(F32)

            def _grp(j4, cc):
                for k in range(8):
                    cs = j4 * 128 + k * 16
                    le = le_v[rr, pl.ds(cs, 16)]
                    dxm = dx_v[rr, pl.ds(cs, 16)]
                    dym = dy_v[rr, pl.ds(cs, 16)]
                    binv = b * (NCLS - 1) + jnp.maximum(le - 1, 0)
                    cx = plsc.load_gather(cx_v, [binv])
                    cy = plsc.load_gather(cy_v, [binv])
                    xf = (cs + lanes).astype(F32)
                    ux = cx - xf
                    uy = cy - yf
                    u2 = jnp.maximum(ux * ux + uy * uy, 1e-12)
                    r2 = _rsqrt(u2)
                    num = ux * dxm + uy * dym
                    vm = num > 0.9 * (u2 * r2 + EPS)
                    one = jnp.ones((16,), F32)
                    plsc.addupdate_scatter(vacc_v,
                                           [lanes * VOTEL + binv],
                                           one, mask=vm)
                return cc
            lax.fori_loop(0, JROW // 8, _grp, 0)
            return carry
        lax.fori_loop(0, 8, _row, 0)
        return 0
    lax.fori_loop(u0, u1, _unit, 0)

    def _foldv(t, _):
        v = vacc_v[pl.ds(t * 16, 16)]
        for l in range(1, 16):
            v = v + vacc_v[pl.ds(l * VOTEL + t * 16, 16)]
        vacc_v[pl.ds(t * 16, 16)] = v
        return 0
    lax.fori_loop(0, VOTESTRIDE // 16, _foldv, 0)
    pltpu.sync_copy(vacc_v.at[pl.ds(0, VOTESTRIDE)],
                    votes_hbm.at[pl.ds(wid * VOTESTRIDE, VOTESTRIDE)])


def _fin_body(sums_ref, votes_ref, ext_ref, poses_ref, meta_ref,
              box_ref, pose_ref):
    sums = sums_ref[...]
    votes = jnp.sum(votes_ref[...], axis=0)[:NBIN]
    cnt = sums[:, 0]
    a11 = sums[:, 1]
    a12 = sums[:, 2]
    a22 = sums[:, 3]
    b1 = sums[:, 4]
    b2 = sums[:, 5]
    sz = sums[:, 6]
    det = a11 * a22 - a12 * a12
    det = jnp.where(jnp.abs(det) < EPS, EPS, det)
    cx = (a22 * b1 - a12 * b2) / det
    cy = (a11 * b2 - a12 * b1) / det

    bsel = lax.broadcasted_iota(jnp.int32, (NBIN,), 0) >= (NCLS - 1)
    fx = jnp.where(bsel, meta_ref[1, 0], meta_ref[0, 0]) + EPS
    fy = jnp.where(bsel, meta_ref[1, 4], meta_ref[0, 4]) + EPS
    px0 = jnp.where(bsel, meta_ref[1, 2], meta_ref[0, 2])
    py0 = jnp.where(bsel, meta_ref[1, 5], meta_ref[0, 5])

    frac = votes / jnp.maximum(cnt, 1.0)
    depth = jnp.exp(sz / jnp.maximum(cnt, 1.0))
    tx = depth * (cx - px0) / fx
    ty = depth * (cy - py0) / fy

    e = ext_ref[...]
    ext2 = jnp.sum(e * e, axis=1)
    e21 = ext2[1:NCLS]
    ext2sel = jnp.concatenate([e21, e21])
    extv = jnp.sqrt(ext2sel + EPS)
    bw = 0.5 * fx * extv / (depth + EPS)
    bh = 0.5 * fy * extv / (depth + EPS)
    validv = ((cnt > 500.0) & (votes >= 100.0)).astype(F32)
    score = frac * validv

    rowi = lax.broadcasted_iota(jnp.int32, (NBIN,), 0)
    clsv = (rowi + 1 - (NCLS - 1) * bsel.astype(jnp.int32)).astype(F32)
    box_ref[0, :] = bsel.astype(F32)
    box_ref[1, :] = clsv
    box_ref[2, :] = cx - bw
    box_ref[3, :] = cy - bh
    box_ref[4, :] = cx + bw
    box_ref[5, :] = cy + bh
    box_ref[6, :] = score

    for k in range(4):
        pose_ref[k, :] = jnp.where(bsel, poses_ref[1, 6 + k],
                                   poses_ref[0, 6 + k])
    pose_ref[4, :] = tx
    pose_ref[5, :] = ty
    pose_ref[6, :] = depth


_mesh = plsc.VectorSubcoreMesh(core_axis_name="c", subcore_axis_name="s")

_moments = pl.kernel(
    _mom_body,
    out_type=[jax.ShapeDtypeStruct((NW * ACCW,), F32)],
    mesh=_mesh,
    compiler_params=pltpu.CompilerParams(needs_layout_passes=False),
    scratch_types=[
        pltpu.VMEM((8, W), jnp.int32),
        pltpu.VMEM((8, W), F32),
        pltpu.VMEM((8, W), F32),
        pltpu.VMEM((8, W), F32),
        pltpu.VMEM((16 * ACCL + 16,), F32),
    ],
)

_votes = pl.kernel(
    _vote_body,
    out_type=[
        jax.ShapeDtypeStruct((NW * VOTESTRIDE,), F32),
        jax.ShapeDtypeStruct((ACCW,), F32),
    ],
    mesh=_mesh,
    compiler_params=pltpu.CompilerParams(needs_layout_passes=False),
    scratch_types=[
        pltpu.VMEM((8, W), jnp.int32),
        pltpu.VMEM((8, W), F32),
        pltpu.VMEM((8, W), F32),
        pltpu.VMEM((NW * ACCW,), F32),
        pltpu.VMEM((VOTESTRIDE,), F32),
        pltpu.VMEM((VOTESTRIDE,), F32),
        pltpu.VMEM((16 * VOTEL + 16,), F32),
    ],
)

_final = pl.pallas_call(
    _fin_body,
    out_shape=[
        jax.ShapeDtypeStruct((7, NBIN), F32),
        jax.ShapeDtypeStruct((7, NBIN), F32),
    ],
)


def kernel(labels, masks, vertex_pred, extents, poses, meta_data):
    lab2 = labels.reshape(RH, W)
    msk2 = masks.reshape(RH, W)
    labe, dxm, dym, vzm = _select(lab2, msk2, vertex_pred)
    part, = _moments(labe, dxm, dym, vzm)
    votes, sums = _votes(labe, dxm, dym, part)
    boxt, poset = _final(sums.reshape(NBIN, 8), votes.reshape(NW, VOTESTRIDE),
                         extents, poses, meta_data)
    top_box = boxt.T
    top_pose = poset.T
    top_target = jnp.zeros((NBIN, 4 * NCLS), F32)
    top_weight = jnp.zeros((NBIN, 4 * NCLS), F32)
    top_domain = jnp.repeat(jnp.arange(B, dtype=F32), NCLS - 1)
    return top_box, top_pose, top_target, top_weight, top_domain


# speedup vs baseline: 1.0837x; 1.0605x over previous
import numpy as np
import jax
import jax.numpy as jnp
from jax import lax
from jax.experimental import pallas as pl
from jax.experimental.pallas import tpu as pltpu
from jax.experimental.pallas import tpu_sc as plsc

B, H, W, NCLS = 2, 480, 640, 22
HW = H * W
NPIX = B * HW
RH = B * H
NC, NS = 2, 16
NW = NC * NS
NUNIT = RH // 8
JROW = W // 16
NBIN = B * (NCLS - 1)
ACCW = NBIN * 8
ACCL = ACCW + 1
VOTESTRIDE = 48
VOTEL = VOTESTRIDE + 1
BH = 16
EPS = 1e-6
F32 = jnp.float32


def _rsqrt(n2):
    i = lax.bitcast_convert_type(n2, jnp.int32)
    i = 0x5F3759DF - (i >> 1)
    r = lax.bitcast_convert_type(i, F32)
    for _ in range(3):
        r = r * (1.5 - 0.5 * n2 * r * r)
    return r


def _sel_body(lab_ref, msk_ref, vp_ref, labe_ref, dxm_ref, dym_ref, vzm_ref):
    lab = lab_ref[...]
    msk = msk_ref[...]
    le = jnp.where((msk > 0) & (lab > 0), lab, 0)
    vx = vp_ref[0, 0]
    vy = vp_ref[0, 1]
    vz = vp_ref[0, 2]
    for c in range(1, NCLS):
        mc = le == c
        vx = jnp.where(mc, vp_ref[0, 3 * c], vx)
        vy = jnp.where(mc, vp_ref[0, 3 * c + 1], vy)
        vz = jnp.where(mc, vp_ref[0, 3 * c + 2], vz)
    valid = le > 0
    n2 = jnp.maximum(vx * vx + vy * vy, 1e-12)
    r = lax.rsqrt(n2)
    s = r / (1.0 + EPS * r)
    zero = jnp.zeros_like(vx)
    labe_ref[...] = le
    dxm_ref[...] = jnp.where(valid, vx * s, zero)
    dym_ref[...] = jnp.where(valid, vy * s, zero)
    vzm_ref[...] = jnp.where(valid, vz, zero)


def _make_select(half):
    hb = H // BH
    out_spec = pl.BlockSpec((BH, W), lambda g: (g, 0))
    return pl.pallas_call(
        _sel_body,
        grid=(hb,),
        in_specs=[
            pl.BlockSpec((BH, W), lambda g: (half * hb + g, 0)),
            pl.BlockSpec((BH, W), lambda g: (half * hb + g, 0)),
            pl.BlockSpec((1, 3 * NCLS, BH, W), lambda g: (half, 0, g, 0)),
        ],
        out_specs=[out_spec, out_spec, out_spec, out_spec],
        out_shape=[
            jax.ShapeDtypeStruct((H, W), jnp.int32),
            jax.ShapeDtypeStruct((H, W), F32),
            jax.ShapeDtypeStruct((H, W), F32),
            jax.ShapeDtypeStruct((H, W), F32),
        ],
    )


_select0 = _make_select(0)
_select1 = _make_select(1)


def _mom_body(half, labe_hbm, dxm_hbm, dym_hbm, vzm_hbm, part_hbm,
              le_v, dx_v, dy_v, vz_v, acc_v):
    cid = lax.axis_index("c")
    sid = lax.axis_index("s")
    wid = cid * NS + sid
    b = half
    hu = NUNIT // 2
    u0 = (wid * hu) // NW
    u1 = ((wid + 1) * hu) // NW
    lanes = lax.iota(jnp.int32, 16)

    def _zero(i, _):
        acc_v[pl.ds(i * 16, 16)] = jnp.zeros((16,), F32)
        return 0
    lax.fori_loop(0, (16 * ACCL + 15) // 16, _zero, 0)

    def _unit(u, _):
        r0 = u * 8
        pltpu.sync_copy(labe_hbm.at[pl.ds(r0, 8)], le_v)
        pltpu.sync_copy(dxm_hbm.at[pl.ds(r0, 8)], dx_v)
        pltpu.sync_copy(dym_hbm.at[pl.ds(r0, 8)], dy_v)
        pltpu.sync_copy(vzm_hbm.at[pl.ds(r0, 8)], vz_v)

        def _row(rr, carry):
            yf = (r0 + rr).astype(F32)

            def _grp(j4, cc):
                for k in range(4):
                    cs = j4 * 64 + k * 16
                    le = le_v[rr, pl.ds(cs, 16)]
                    dx = dx_v[rr, pl.ds(cs, 16)]
                    dy = dy_v[rr, pl.ds(cs, 16)]
                    vz = vz_v[rr, pl.ds(cs, 16)]
                    valid = le > 0
                    xf = (cs + lanes).astype(F32)
                    t1 = 1.0 - dx * dx
                    t2 = -dx * dy
                    t3 = 1.0 - dy * dy
                    s1 = t1 * xf + t2 * yf
                    s2 = t2 * xf + t3 * yf
                    binv = b * (NCLS - 1) + jnp.maximum(le - 1, 0)
                    ab = lanes * ACCL + binv * 8
                    one = jnp.ones((16,), F32)
                    plsc.addupdate_scatter(acc_v, [ab], one, mask=valid)
                    plsc.addupdate_scatter(acc_v, [ab + 1], t1, mask=valid)
                    plsc.addupdate_scatter(acc_v, [ab + 2], t2, mask=valid)
                    plsc.addupdate_scatter(acc_v, [ab + 3], t3, mask=valid)
                    plsc.addupdate_scatter(acc_v, [ab + 4], s1, mask=valid)
                    plsc.addupdate_scatter(acc_v, [ab + 5], s2, mask=valid)
                    plsc.addupdate_scatter(acc_v, [ab + 6], vz, mask=valid)
                return cc
            lax.fori_loop(0, JROW // 4, _grp, 0)
            return carry
        lax.fori_loop(0, 8, _row, 0)
        return 0
    lax.fori_loop(u0, u1, _unit, 0)

    def _fold(t, _):
        v = acc_v[pl.ds(t * 16, 16)]
        for l in range(1, 16):
            v = v + acc_v[pl.ds(l * ACCL + t * 16, 16)]
        acc_v[pl.ds(t * 16, 16)] = v
        return 0
    lax.fori_loop(0, ACCW // 16, _fold, 0)
    pltpu.sync_copy(acc_v.at[pl.ds(0, ACCW)],
                    part_hbm.at[pl.ds(wid * ACCW, ACCW)])


def _vote_body(labe0_hbm, labe1_hbm, dxm0_hbm, dxm1_hbm,
               dym0_hbm, dym1_hbm, part0_hbm, part1_hbm,
               votes_hbm, sums_hbm,
               le_v, dx_v, dy_v, part_v, cx_v, cy_v, vacc_v):
    cid = lax.axis_index("c")
    sid = lax.axis_index("s")
    wid = cid * NS + sid
    b = wid // NS
    hu = NUNIT // 2
    lanes = lax.iota(jnp.int32, 16)

    pltpu.sync_copy(part0_hbm, part_v.at[pl.ds(0, NW * ACCW)])
    pltpu.sync_copy(part1_hbm, part_v.at[pl.ds(NW * ACCW, NW * ACCW)])

    def _red(t, _):
        v = part_v[pl.ds(t * 16, 16)]
        for wk in range(1, 2 * NW):
            v = v + part_v[pl.ds(wk * ACCW + t * 16, 16)]
        part_v[pl.ds(t * 16, 16)] = v
        return 0
    lax.fori_loop(0, ACCW // 16, _red, 0)

    @pl.when(wid == 0)
    def _():
        pltpu.sync_copy(part_v.at[pl.ds(0, ACCW)], sums_hbm)

    def _slv(t, _):
        binv = t * 16 + lanes
        bi8 = binv * 8
        a11 = plsc.load_gather(part_v, [bi8 + 1])
        a12 = plsc.load_gather(part_v, [bi8 + 2])
        a22 = plsc.load_gather(part_v, [bi8 + 3])
        b1 = plsc.load_gather(part_v, [bi8 + 4])
        b2 = plsc.load_gather(part_v, [bi8 + 5])
        det = a11 * a22 - a12 * a12
        det = jnp.where(jnp.abs(det) < EPS, jnp.full((16,), EPS, F32), det)
        cx_v[pl.ds(t * 16, 16)] = (a22 * b1 - a12 * b2) / det
        cy_v[pl.ds(t * 16, 16)] = (a11 * b2 - a12 * b1) / det
        return 0
    lax.fori_loop(0, 3, _slv, 0)

    def _zero(i, _):
        vacc_v[pl.ds(i * 16, 16)] = jnp.zeros((16,), F32)
        return 0
    lax.fori_loop(0, (16 * VOTEL + 15) // 16, _zero, 0)

    wm = wid - b * NS
    ug0 = (wm * hu) // NS
    ug1 = ((wm + 1) * hu) // NS

    def _half(labe_hbm, dxm_hbm, dym_hbm):
        def _unit(u, _):
            r0 = u * 8
            pltpu.sync_copy(labe_hbm.at[pl.ds(r0, 8)], le_v)
            pltpu.sync_copy(dxm_hbm.at[pl.ds(r0, 8)], dx_v)
            pltpu.sync_copy(dym_hbm.at[pl.ds(r0, 8)], dy_v)

            def _row(rr, carry):
                yf = (r0 + rr).astype(F32)

                def _grp(j4, cc):
                    for k in range(8):
                        cs = j4 * 128 + k * 16
                        le = le_v[rr, pl.ds(cs, 16)]
                        dxm = dx_v[rr, pl.ds(cs, 16)]
                        dym = dy_v[rr, pl.ds(cs, 16)]
                        binv = b * (NCLS - 1) + jnp.maximum(le - 1, 0)
                        cx = plsc.load_gather(cx_v, [binv])
                        cy = plsc.load_gather(cy_v, [binv])
                        xf = (cs + lanes).astype(F32)
                        ux = cx - xf
                        uy = cy - yf
                        u2 = jnp.maximum(ux * ux + uy * uy, 1e-12)
                        r2 = _rsqrt(u2)
                        num = ux * dxm + uy * dym
                        vm = num > 0.9 * (u2 * r2 + EPS)
                        one = jnp.ones((16,), F32)
                        plsc.addupdate_scatter(vacc_v,
                                               [lanes * VOTEL + binv],
                                               one, mask=vm)
                    return cc
                lax.fori_loop(0, JROW // 8, _grp, 0)
                return carry
            lax.fori_loop(0, 8, _row, 0)
            return 0
        lax.fori_loop(ug0, ug1, _unit, 0)

    @pl.when(wid < NS)
    def _():
        _half(labe0_hbm, dxm0_hbm, dym0_hbm)

    @pl.when(wid >= NS)
    def _():
        _half(labe1_hbm, dxm1_hbm, dym1_hbm)

    def _foldv(t, _):
        v = vacc_v[pl.ds(t * 16, 16)]
        for l in range(1, 16):
            v = v + vacc_v[pl.ds(l * VOTEL + t * 16, 16)]
        vacc_v[pl.ds(t * 16, 16)] = v
        return 0
    lax.fori_loop(0, VOTESTRIDE // 16, _foldv, 0)
    pltpu.sync_copy(vacc_v.at[pl.ds(0, VOTESTRIDE)],
                    votes_hbm.at[pl.ds(wid * VOTESTRIDE, VOTESTRIDE)])


def _fin_body(sums_ref, votes_ref, ext_ref, poses_ref, meta_ref,
              box_ref, pose_ref):
    sums = sums_ref[...]
    votes = jnp.sum(votes_ref[...], axis=0)[:NBIN]
    cnt = sums[:, 0]
    a11 = sums[:, 1]
    a12 = sums[:, 2]
    a22 = sums[:, 3]
    b1 = sums[:, 4]
    b2 = sums[:, 5]
    sz = sums[:, 6]
    det = a11 * a22 - a12 * a12
    det = jnp.where(jnp.abs(det) < EPS, EPS, det)
    cx = (a22 * b1 - a12 * b2) / det
    cy = (a11 * b2 - a12 * b1) / det

    bsel = lax.broadcasted_iota(jnp.int32, (NBIN,), 0) >= (NCLS - 1)
    fx = jnp.where(bsel, meta_ref[1, 0], meta_ref[0, 0]) + EPS
    fy = jnp.where(bsel, meta_ref[1, 4], meta_ref[0, 4]) + EPS
    px0 = jnp.where(bsel, meta_ref[1, 2], meta_ref[0, 2])
    py0 = jnp.where(bsel, meta_ref[1, 5], meta_ref[0, 5])

    frac = votes / jnp.maximum(cnt, 1.0)
    depth = jnp.exp(sz / jnp.maximum(cnt, 1.0))
    tx = depth * (cx - px0) / fx
    ty = depth * (cy - py0) / fy

    e = ext_ref[...]
    ext2 = jnp.sum(e * e, axis=1)
    e21 = ext2[1:NCLS]
    ext2sel = jnp.concatenate([e21, e21])
    extv = jnp.sqrt(ext2sel + EPS)
    bw = 0.5 * fx * extv / (depth + EPS)
    bh = 0.5 * fy * extv / (depth + EPS)
    validv = ((cnt > 500.0) & (votes >= 100.0)).astype(F32)
    score = frac * validv

    rowi = lax.broadcasted_iota(jnp.int32, (NBIN,), 0)
    clsv = (rowi + 1 - (NCLS - 1) * bsel.astype(jnp.int32)).astype(F32)
    box_ref[0, :] = bsel.astype(F32)
    box_ref[1, :] = clsv
    box_ref[2, :] = cx - bw
    box_ref[3, :] = cy - bh
    box_ref[4, :] = cx + bw
    box_ref[5, :] = cy + bh
    box_ref[6, :] = score

    for k in range(4):
        pose_ref[k, :] = jnp.where(bsel, poses_ref[1, 6 + k],
                                   poses_ref[0, 6 + k])
    pose_ref[4, :] = tx
    pose_ref[5, :] = ty
    pose_ref[6, :] = depth


_mesh = plsc.VectorSubcoreMesh(core_axis_name="c", subcore_axis_name="s")

def _make_moments(half):
    import functools
    return pl.kernel(
        functools.partial(_mom_body, half),
        out_type=[jax.ShapeDtypeStruct((NW * ACCW,), F32)],
        mesh=_mesh,
        compiler_params=pltpu.CompilerParams(needs_layout_passes=False),
        scratch_types=[
            pltpu.VMEM((8, W), jnp.int32),
            pltpu.VMEM((8, W), F32),
            pltpu.VMEM((8, W), F32),
            pltpu.VMEM((8, W), F32),
            pltpu.VMEM((16 * ACCL + 16,), F32),
        ],
    )


_moments0 = _make_moments(0)
_moments1 = _make_moments(1)

_votes = pl.kernel(
    _vote_body,
    out_type=[
        jax.ShapeDtypeStruct((NW * VOTESTRIDE,), F32),
        jax.ShapeDtypeStruct((ACCW,), F32),
    ],
    mesh=_mesh,
    compiler_params=pltpu.CompilerParams(needs_layout_passes=False),
    scratch_types=[
        pltpu.VMEM((8, W), jnp.int32),
        pltpu.VMEM((8, W), F32),
        pltpu.VMEM((8, W), F32),
        pltpu.VMEM((2 * NW * ACCW,), F32),
        pltpu.VMEM((VOTESTRIDE,), F32),
        pltpu.VMEM((VOTESTRIDE,), F32),
        pltpu.VMEM((16 * VOTEL + 16,), F32),
    ],
)

_final = pl.pallas_call(
    _fin_body,
    out_shape=[
        jax.ShapeDtypeStruct((7, NBIN), F32),
        jax.ShapeDtypeStruct((7, NBIN), F32),
    ],
)


def kernel(labels, masks, vertex_pred, extents, poses, meta_data):
    lab2 = labels.reshape(RH, W)
    msk2 = masks.reshape(RH, W)
    labe0, dxm0, dym0, vzm0 = _select0(lab2, msk2, vertex_pred)
    labe1, dxm1, dym1, vzm1 = _select1(lab2, msk2, vertex_pred)
    part0, = _moments0(labe0, dxm0, dym0, vzm0)
    part1, = _moments1(labe1, dxm1, dym1, vzm1)
    votes, sums = _votes(labe0, labe1, dxm0, dxm1, dym0, dym1, part0, part1)
    boxt, poset = _final(sums.reshape(NBIN, 8), votes.reshape(NW, VOTESTRIDE),
                         extents, poses, meta_data)
    top_box = boxt.T
    top_pose = poset.T
    top_target = jnp.zeros((NBIN, 4 * NCLS), F32)
    top_weight = jnp.zeros((NBIN, 4 * NCLS), F32)
    top_domain = jnp.repeat(jnp.arange(B, dtype=F32), NCLS - 1)
    return top_box, top_pose, top_target, top_weight, top_domain
